# baseline (speedup 1.0000x reference)
import functools

import jax
import jax.numpy as jnp
from jax import lax
from jax.experimental import pallas as pl
from jax.experimental.pallas import tpu as pltpu
from jax.experimental.pallas import tpu_sc as plsc

ALPHA = 8
FRAMES = 32


def _fast_body(x_ref, fast_ref):
    fast_ref[...] = x_ref[...]


_SC_MESH = plsc.VectorSubcoreMesh(core_axis_name="c", subcore_axis_name="s")


def _slow_body(x_hbm, slow_hbm, buf):
    wid = lax.axis_index("s") * 2 + lax.axis_index("c")

    @pl.when(wid < 24)
    def _():
        ch = wid // ALPHA
        s = wid % ALPHA
        pltpu.sync_copy(x_hbm.at[ch, 9 * s], buf)
        pltpu.sync_copy(buf, slow_hbm.at[ch, s])


def kernel(x):
    C, T, H, W = x.shape
    G = T // ALPHA
    slow_fn = pl.kernel(
        _slow_body,
        out_type=jax.ShapeDtypeStruct((C, G, H, W), x.dtype),
        mesh=_SC_MESH,
        scratch_types=[pltpu.VMEM((H, W), x.dtype)],
    )
    slow = slow_fn(x)
    fast = pl.pallas_call(
        _fast_body,
        grid=(C, T // FRAMES),
        in_specs=[pl.BlockSpec((1, FRAMES, H, W), lambda c, h: (c, h, 0, 0))],
        out_specs=pl.BlockSpec((1, FRAMES, H, W), lambda c, h: (c, h, 0, 0)),
        out_shape=jax.ShapeDtypeStruct((C, T, H, W), x.dtype),
    )(x)
    return (slow, fast)


# speedup vs baseline: 1.0005x; 1.0005x over previous
import functools

import jax
import jax.numpy as jnp
from jax import lax
from jax.experimental import pallas as pl
from jax.experimental.pallas import tpu as pltpu
from jax.experimental.pallas import tpu_sc as plsc

ALPHA = 8
FRAMES = 32


def _fast_body(x_ref, fast_ref):
    fast_ref[...] = x_ref[...]


_SC_MESH = plsc.VectorSubcoreMesh(core_axis_name="c", subcore_axis_name="s")


def _slow_body(x_hbm, slow_hbm, buf):
    wid = lax.axis_index("s") * 2 + lax.axis_index("c")

    @pl.when(wid < 24)
    def _():
        ch = wid // ALPHA
        s = wid % ALPHA
        pltpu.sync_copy(x_hbm.at[ch, 9 * s], buf)
        pltpu.sync_copy(buf, slow_hbm.at[ch, s])


def kernel(x):
    C, T, H, W = x.shape
    G = T // ALPHA
    slow_fn = pl.kernel(
        _slow_body,
        out_type=jax.ShapeDtypeStruct((C, G, H, W), x.dtype),
        mesh=_SC_MESH,
        scratch_types=[pltpu.VMEM((H, W), x.dtype)],
        cost_estimate=pl.CostEstimate(
            flops=0,
            transcendentals=0,
            bytes_accessed=2 * C * G * H * W * 4,
        ),
    )
    slow = slow_fn(x)
    fast = pl.pallas_call(
        _fast_body,
        grid=(C, T // FRAMES),
        in_specs=[pl.BlockSpec((1, FRAMES, H, W), lambda c, h: (c, h, 0, 0))],
        out_specs=pl.BlockSpec((1, FRAMES, H, W), lambda c, h: (c, h, 0, 0)),
        out_shape=jax.ShapeDtypeStruct((C, T, H, W), x.dtype),
    )(x)
    return (slow, fast)


# speedup vs baseline: 1.6420x; 1.6412x over previous
import jax
import jax.numpy as jnp
from jax.experimental import pallas as pl

ALPHA = 8
FRAMES = 16


def _pack_body(x_ref, slow_ref, fast_ref):
    h = pl.program_id(0)
    fast_ref[...] = x_ref[...]
    slow_ref[:, 0] = x_ref[:, 2 * h]
    slow_ref[:, 1] = x_ref[:, 2 * h + 9]


def kernel(x):
    C, T, H, W = x.shape
    G = T // ALPHA
    slow, fast = pl.pallas_call(
        _pack_body,
        grid=(T // FRAMES,),
        in_specs=[pl.BlockSpec((C, FRAMES, H, W), lambda h: (0, h, 0, 0))],
        out_specs=[
            pl.BlockSpec((C, 2, H, W), lambda h: (0, h, 0, 0)),
            pl.BlockSpec((C, FRAMES, H, W), lambda h: (0, h, 0, 0)),
        ],
        out_shape=[
            jax.ShapeDtypeStruct((C, G, H, W), x.dtype),
            jax.ShapeDtypeStruct((C, T, H, W), x.dtype),
        ],
    )(x)
    return (slow, fast)
